# baseline (speedup 1.0000x reference)
import functools

import jax
import jax.numpy as jnp
from jax import lax
from jax.experimental import pallas as pl
from jax.experimental.pallas import tpu as pltpu
from jax.experimental.pallas import tpu_sc as plsc

B = 16384
SEQ = 50
PADS = 64
E = 32
NCLS = 20

_info = plsc.get_sparse_core_info()
NC, NS = _info.num_cores, _info.num_subcores
NW = NC * NS
RPW = B // NW
CHUNK_ROWS = 2
CHUNK_IDX = CHUNK_ROWS * PADS
NCHUNK = RPW // CHUNK_ROWS
IPW = RPW * PADS


def _tree_sum(loads):
    parts = []
    for k in range(4):
        chain = loads[k::4]
        acc = chain[0]
        for v in chain[1:]:
            acc = acc + v
        parts.append(acc)
    return (parts[0] + parts[1]) + (parts[2] + parts[3])


def _sc_pool_body(ids_hbm, table_hbm, out_hbm, idx_v, rows_v, feat_v, sem):
    wid = lax.axis_index("s") * NC + lax.axis_index("c")
    base_row = wid * RPW
    base_idx = wid * IPW

    pltpu.sync_copy(ids_hbm.at[pl.ds(base_idx, IPW)], idx_v)

    def chunk(c, _):
        pltpu.async_copy(
            table_hbm.at[idx_v.at[pl.ds(c * CHUNK_IDX, CHUNK_IDX)]],
            rows_v, sem).wait()
        inv = jnp.float32(1.0 / SEQ)
        for r in range(CHUNK_ROWS):
            for h in range(2):
                loads = [rows_v[r * PADS + s, pl.ds(16 * h, 16)]
                         for s in range(PADS)]
                feat_v[c * CHUNK_ROWS + r, pl.ds(16 * h, 16)] = (
                    _tree_sum(loads) * inv)
        return 0

    lax.fori_loop(0, NCHUNK, chunk, 0)
    pltpu.sync_copy(feat_v, out_hbm.at[pl.ds(base_row, RPW)])


@functools.partial(
    pl.kernel,
    out_type=jax.ShapeDtypeStruct((B, E), jnp.float32),
    mesh=plsc.VectorSubcoreMesh(core_axis_name="c", subcore_axis_name="s"),
    scratch_types=[
        pltpu.VMEM((IPW,), jnp.int32),
        pltpu.VMEM((CHUNK_IDX, E), jnp.float32),
        pltpu.VMEM((RPW, E), jnp.float32),
        pltpu.SemaphoreType.DMA,
    ],
    compiler_params=pltpu.CompilerParams(use_tc_tiling_on_sc=False),
)
def _sc_pool(ids_hbm, table_hbm, out_hbm, idx_v, rows_v, feat_v, sem):
    _sc_pool_body(ids_hbm, table_hbm, out_hbm, idx_v, rows_v, feat_v, sem)


def _mm_body(f_ref, w_ref, b_ref, o_ref):
    o_ref[...] = (
        lax.dot_general(f_ref[...], w_ref[...],
                        (((1,), (1,)), ((), ())),
                        preferred_element_type=jnp.float32)
        + b_ref[...])


_MM_BLK = 1024


def _tc_logits(feats, W, b2d):
    return pl.pallas_call(
        _mm_body,
        grid=(B // _MM_BLK,),
        in_specs=[
            pl.BlockSpec((_MM_BLK, E), lambda i: (i, 0)),
            pl.BlockSpec((NCLS, E), lambda i: (0, 0)),
            pl.BlockSpec((1, NCLS), lambda i: (0, 0)),
        ],
        out_specs=pl.BlockSpec((_MM_BLK, NCLS), lambda i: (i, 0)),
        out_shape=jax.ShapeDtypeStruct((B, NCLS), jnp.float32),
    )(feats, W, b2d)


def kernel(input_ids, table, W, b):
    ids = input_ids.astype(jnp.int32)
    ids_pad = jnp.zeros((B, PADS), jnp.int32).at[:, :SEQ].set(ids)
    feats = _sc_pool(ids_pad.reshape(-1), table)
    return _tc_logits(feats, W, b.reshape(1, NCLS))
